# baseline (speedup 1.0000x reference)
import jax
import jax.numpy as jnp
from jax.experimental import pallas as pl
from jax.experimental.pallas import tpu as pltpu
from jax.experimental.pallas import tpu_sc as plsc

DIM_OUT = 1000
KB = 8
LB = 1024
VEC = 16


def _sc_body(xcol, kv, o_vmem):
    @pl.loop(0, KB)
    def _(kk):
        k = kv[pl.ds(kk, 1), 0][0]

        @pl.loop(0, LB, step=VEC)
        def _(l):
            vec = xcol.at[0, pl.ds(l, VEC)][...]
            o_vmem.at[kk, pl.ds(l, VEC)][...] = jnp.where(
                vec == k, jnp.float32(1.0), jnp.float32(0.0)
            )


def kernel(x):
    x = x.astype(jnp.int32)
    B, C = x.shape
    xt = x.T
    kvals = jax.lax.broadcasted_iota(jnp.int32, (C * DIM_OUT, 1), 0) % DIM_OUT

    mesh = plsc.VectorSubcoreMesh(core_axis_name="core", subcore_axis_name="subcore")

    @pl.kernel(
        out_type=jax.ShapeDtypeStruct((C * DIM_OUT, B), jnp.float32),
        mesh=mesh,
    )
    def sc_onehot(xt_hbm, kv_hbm, o_hbm):
        def body(xcol_vmem, kv_vmem, o_vmem):
            _sc_body(xcol_vmem, kv_vmem, o_vmem)

        pltpu.emit_pipeline(
            body,
            grid=(C * DIM_OUT // KB, B // LB),
            in_specs=[
                pl.BlockSpec((1, LB), index_map=lambda m, l: ((m * KB) // DIM_OUT, l)),
                pl.BlockSpec((KB, 1), index_map=lambda m, l: (m, 0)),
            ],
            out_specs=[pl.BlockSpec((KB, LB), index_map=lambda m, l: (m, l))],
            core_axis_name=("core", "subcore"),
            dimension_semantics=(pltpu.PARALLEL, pltpu.PARALLEL),
        )(xt_hbm, kv_hbm, o_hbm)

    out_flat = sc_onehot(xt, kvals)
    return jnp.transpose(out_flat.reshape(C, DIM_OUT, B), (2, 0, 1))


# speedup vs baseline: 7.6312x; 7.6312x over previous
import jax
import jax.numpy as jnp
from jax.experimental import pallas as pl
from jax.experimental.pallas import tpu as pltpu

DIM_OUT = 1000
KBLK = 200
NBUF = 16
KCH = DIM_OUT // KBLK


def _onehot_kernel(x_ref, o_hbm, stage, sem):
    C = x_ref.shape[0]
    B = x_ref.shape[2]
    nchunks = C * KCH

    def chunk_copy(i, b):
        c = jax.lax.div(i, KCH)
        k = jax.lax.rem(i, KCH)
        return pltpu.make_async_copy(
            stage.at[b],
            o_hbm.at[pl.ds(c, 1), pl.ds(k * KBLK, KBLK), :],
            sem.at[b],
        )

    iota = jax.lax.broadcasted_iota(jnp.int32, (1, KBLK, 1), 1)

    def body(i, _):
        b = jax.lax.rem(i, NBUF)

        @pl.when(i >= NBUF)
        def _():
            chunk_copy(i - NBUF, b).wait()

        c = jax.lax.div(i, KCH)
        k = jax.lax.rem(i, KCH)
        idx = x_ref[pl.ds(c, 1), :, :]
        stage.at[b][...] = (idx == iota + k * KBLK).astype(jnp.float32)
        chunk_copy(i, b).start()
        return 0

    jax.lax.fori_loop(0, nchunks, body, 0)

    def drain(i, _):
        chunk_copy(i, jax.lax.rem(i, NBUF)).wait()
        return 0

    jax.lax.fori_loop(nchunks - NBUF, nchunks, drain, 0)


def kernel(x):
    x = x.astype(jnp.int32)
    B, C = x.shape
    xt = x.T.reshape(C, 1, B)
    out_t = pl.pallas_call(
        _onehot_kernel,
        in_specs=[pl.BlockSpec(memory_space=pltpu.VMEM)],
        out_specs=pl.BlockSpec(memory_space=pl.ANY),
        out_shape=jax.ShapeDtypeStruct((C, DIM_OUT, B), jnp.float32),
        scratch_shapes=[
            pltpu.VMEM((NBUF, 1, KBLK, B), jnp.float32),
            pltpu.SemaphoreType.DMA((NBUF,)),
        ],
    )(xt)
    return jnp.transpose(out_t, (2, 0, 1))


# speedup vs baseline: 7.6333x; 1.0003x over previous
import jax
import jax.numpy as jnp
from jax.experimental import pallas as pl
from jax.experimental.pallas import tpu as pltpu

DIM_OUT = 1000
KBLK = 200
NBUF = 12
KCH = DIM_OUT // KBLK


def _onehot_kernel(x_ref, o_hbm, stage, sem):
    C = x_ref.shape[0]
    B = x_ref.shape[2]
    nchunks = C * KCH

    def chunk_copy(i, b):
        c = jax.lax.div(i, KCH)
        k = jax.lax.rem(i, KCH)
        return pltpu.make_async_copy(
            stage.at[b],
            o_hbm.at[pl.ds(c, 1), pl.ds(k * KBLK, KBLK), :],
            sem.at[b],
        )

    iota = jax.lax.broadcasted_iota(jnp.int32, (1, KBLK, 1), 1)

    def body(i, _):
        b = jax.lax.rem(i, NBUF)

        @pl.when(i >= NBUF)
        def _():
            chunk_copy(i - NBUF, b).wait()

        c = jax.lax.div(i, KCH)
        k = jax.lax.rem(i, KCH)
        idx = x_ref[pl.ds(c, 1), :, :]
        stage.at[b][...] = (idx == iota + k * KBLK).astype(jnp.float32)
        chunk_copy(i, b).start()
        return 0

    jax.lax.fori_loop(0, nchunks, body, 0)

    def drain(i, _):
        chunk_copy(i, jax.lax.rem(i, NBUF)).wait()
        return 0

    jax.lax.fori_loop(nchunks - NBUF, nchunks, drain, 0)


def kernel(x):
    x = x.astype(jnp.int32)
    B, C = x.shape
    xt = x.T.reshape(C, 1, B)
    out_t = pl.pallas_call(
        _onehot_kernel,
        in_specs=[pl.BlockSpec(memory_space=pltpu.VMEM)],
        out_specs=pl.BlockSpec(memory_space=pl.ANY),
        out_shape=jax.ShapeDtypeStruct((C, DIM_OUT, B), jnp.float32),
        scratch_shapes=[
            pltpu.VMEM((NBUF, 1, KBLK, B), jnp.float32),
            pltpu.SemaphoreType.DMA((NBUF,)),
        ],
    )(xt)
    return jnp.transpose(out_t, (2, 0, 1))


# speedup vs baseline: 7.6360x; 1.0004x over previous
import jax
import jax.numpy as jnp
from jax.experimental import pallas as pl
from jax.experimental.pallas import tpu as pltpu

DIM_OUT = 1000
KBLK = 200
NBUF = 14
KCH = DIM_OUT // KBLK


def _onehot_kernel(x_ref, o_hbm, stage, sem):
    C = x_ref.shape[0]
    B = x_ref.shape[2]
    nchunks = C * KCH

    def chunk_copy(i, b):
        c = jax.lax.div(i, KCH)
        k = jax.lax.rem(i, KCH)
        return pltpu.make_async_copy(
            stage.at[b],
            o_hbm.at[pl.ds(c, 1), pl.ds(k * KBLK, KBLK), :],
            sem.at[b],
        )

    iota = jax.lax.broadcasted_iota(jnp.int32, (1, KBLK, 1), 1)

    def body(i, _):
        b = jax.lax.rem(i, NBUF)

        @pl.when(i >= NBUF)
        def _():
            chunk_copy(i - NBUF, b).wait()

        c = jax.lax.div(i, KCH)
        k = jax.lax.rem(i, KCH)
        idx = x_ref[pl.ds(c, 1), :, :]
        stage.at[b][...] = (idx == iota + k * KBLK).astype(jnp.float32)
        chunk_copy(i, b).start()
        return 0

    jax.lax.fori_loop(0, nchunks, body, 0)

    def drain(i, _):
        chunk_copy(i, jax.lax.rem(i, NBUF)).wait()
        return 0

    jax.lax.fori_loop(nchunks - NBUF, nchunks, drain, 0)


def kernel(x):
    x = x.astype(jnp.int32)
    B, C = x.shape
    xt = x.T.reshape(C, 1, B)
    out_t = pl.pallas_call(
        _onehot_kernel,
        in_specs=[pl.BlockSpec(memory_space=pltpu.VMEM)],
        out_specs=pl.BlockSpec(memory_space=pl.ANY),
        out_shape=jax.ShapeDtypeStruct((C, DIM_OUT, B), jnp.float32),
        scratch_shapes=[
            pltpu.VMEM((NBUF, 1, KBLK, B), jnp.float32),
            pltpu.SemaphoreType.DMA((NBUF,)),
        ],
    )(xt)
    return jnp.transpose(out_t, (2, 0, 1))
